# baseline (speedup 1.0000x reference)
import functools

import jax
import jax.numpy as jnp
from jax import lax
from jax.experimental import pallas as pl
from jax.experimental.pallas import tpu as pltpu
from jax.experimental.pallas import tpu_sc as plsc

CHUNK = 128


@functools.cache
def _build(B, D, NC, NS):
    NW = NC * NS
    rows_per_w = B // NW
    n_chunks = rows_per_w // CHUNK
    mesh = plsc.VectorSubcoreMesh(core_axis_name="c", subcore_axis_name="s")

    @functools.partial(
        pl.kernel,
        out_type=jax.ShapeDtypeStruct((B, D), jnp.float32),
        mesh=mesh,
        scratch_types=[
            pltpu.VMEM((n_chunks, CHUNK), jnp.int32),
            pltpu.VMEM((CHUNK, D), jnp.float32),
            pltpu.SemaphoreType.DMA,
        ],
        compiler_params=pltpu.CompilerParams(use_tc_tiling_on_sc=False),
    )
    def k(table_hbm, idx_hbm, out_hbm, idx_v, rows_v, sem):
        wid = lax.axis_index("s") * NC + lax.axis_index("c")
        base = wid * rows_per_w
        pltpu.sync_copy(idx_hbm.at[wid], idx_v)

        def body(j, carry):
            pltpu.async_copy(table_hbm.at[idx_v.at[j]], rows_v, sem).wait()
            pltpu.sync_copy(rows_v, out_hbm.at[pl.ds(base + j * CHUNK, CHUNK)])
            return carry

        lax.fori_loop(0, n_chunks, body, 0)

    return k


def kernel(indices, table):
    S0, S1 = indices.shape
    B = S0 * S1
    D = table.shape[1]
    info = plsc.get_sparse_core_info()
    NC, NS = info.num_cores, info.num_subcores
    NW = NC * NS
    idx = indices.astype(jnp.int32).reshape(NW, (B // NW) // CHUNK, CHUNK)
    out = _build(B, D, NC, NS)(table, idx)
    return out.reshape(S0, S1, D)


# speedup vs baseline: 1.1126x; 1.1126x over previous
import functools

import jax
import jax.numpy as jnp
from jax import lax
from jax.experimental import pallas as pl
from jax.experimental.pallas import tpu as pltpu
from jax.experimental.pallas import tpu_sc as plsc

LANE = 128
NBUF = 8
LA = 4


@functools.cache
def _build(B, D, NC, NS):
    NW = NC * NS
    rows_per_w = B // NW
    M = rows_per_w // LANE
    R = M // NBUF if M % NBUF >= LA else M // NBUF - 1
    assert R >= 2 and M - R * NBUF >= LA
    mesh = plsc.VectorSubcoreMesh(core_axis_name="c", subcore_axis_name="s")

    @functools.partial(
        pl.kernel,
        out_type=jax.ShapeDtypeStruct((B, D), jnp.float32),
        mesh=mesh,
        scratch_types=[
            pltpu.VMEM((M, LANE), jnp.int32),
            pltpu.VMEM((NBUF, LANE, D), jnp.float32),
            [pltpu.SemaphoreType.DMA] * NBUF,
            [pltpu.SemaphoreType.DMA] * NBUF,
        ],
        compiler_params=pltpu.CompilerParams(use_tc_tiling_on_sc=False),
    )
    def k(table_hbm, idx_hbm, out_hbm, idx_v, bufs, gsems, wsems):
        wid = lax.axis_index("s") * NC + lax.axis_index("c")
        row_base = wid * rows_per_w

        pltpu.sync_copy(idx_hbm.at[wid], idx_v)

        def issue_g(j, b):
            pltpu.async_copy(table_hbm.at[idx_v.at[j]], bufs.at[b], gsems[b])

        def wait_g(b):
            pltpu.make_async_copy(
                table_hbm.at[idx_v.at[0]], bufs.at[b], gsems[b]
            ).wait()

        def issue_w(j, b):
            pltpu.async_copy(
                bufs.at[b], out_hbm.at[pl.ds(row_base + j * LANE, LANE)], wsems[b]
            )

        def wait_w(b):
            pltpu.make_async_copy(
                bufs.at[b], out_hbm.at[pl.ds(0, LANE)], wsems[b]
            ).wait()

        for b in range(LA):
            issue_g(b, b)

        for b in range(NBUF):
            wait_g(b)
            issue_w(b, b)
            bg = (b + LA) % NBUF
            if b + LA >= NBUF:
                wait_w(bg)
            issue_g(b + LA, bg)

        def round_body(i, carry):
            j0 = i * NBUF
            for b in range(NBUF):
                wait_g(b)
                issue_w(j0 + b, b)
                bg = (b + LA) % NBUF
                wait_w(bg)
                issue_g(j0 + b + LA, bg)
            return carry

        lax.fori_loop(1, R, round_body, 0)

        for j in range(R * NBUF, M):
            b = j % NBUF
            wait_g(b)
            issue_w(j, b)
            if j + LA <= M - 1:
                bg = (j + LA) % NBUF
                wait_w(bg)
                issue_g(j + LA, bg)

        for b in range(NBUF):
            wait_w(b)

    return k


def kernel(indices, table):
    S0, S1 = indices.shape
    B = S0 * S1
    D = table.shape[1]
    info = plsc.get_sparse_core_info()
    NC, NS = info.num_cores, info.num_subcores
    NW = NC * NS
    idx = indices.astype(jnp.int32).reshape(NW, B // (NW * LANE), LANE)
    out = _build(B, D, NC, NS)(table, idx)
    return out.reshape(S0, S1, D)
